# baseline (speedup 1.0000x reference)
import jax
import jax.numpy as jnp
from jax.experimental import pallas as pl
from jax.experimental.pallas import tpu as pltpu

BATCH = 4096
FEAT = 128
NROWS = 100000
BLOCK_N = 1024


def _mm_body(params_ref, x_ref, m_ref, o_ref):
    inv_t = 1.0 / params_ref[0]
    mb = m_ref[...].astype(jnp.bfloat16)
    acc = jax.lax.dot_general(
        x_ref[...], mb,
        dimension_numbers=(((1,), (1,)), ((), ())),
        preferred_element_type=jnp.float32,
    )
    o_ref[...] = acc * inv_t


def kernel(x, y, memory, params):
    del y
    xb = x.astype(jnp.bfloat16)
    grid = (pl.cdiv(NROWS, BLOCK_N),)
    out = pl.pallas_call(
        _mm_body,
        grid=grid,
        in_specs=[
            pl.BlockSpec(memory_space=pltpu.SMEM),
            pl.BlockSpec((BATCH, FEAT), lambda j: (0, 0)),
            pl.BlockSpec((BLOCK_N, FEAT), lambda j: (j, 0)),
        ],
        out_specs=pl.BlockSpec((BATCH, BLOCK_N), lambda j: (0, j)),
        out_shape=jax.ShapeDtypeStruct((BATCH, NROWS), jnp.float32),
    )(params, xb, memory)
    return out
